# Initial kernel scaffold; baseline (speedup 1.0000x reference)
import jax
import jax.numpy as jnp
from jax.experimental import pallas as pl


def kernel(z):
    raise NotImplementedError("write your pallas kernel here")


# speedup vs baseline: 26.7054x; 26.7054x over previous
import functools

import jax
import jax.numpy as jnp
from jax.experimental import pallas as pl

_TEMP = 0.5
_K = 10


def _norm_kernel(z_ref, zn_ref):
    z = z_ref[...]
    s = jnp.sum(z * z, axis=1, keepdims=True)
    zn_ref[...] = z * jax.lax.rsqrt(s)


def _loss_kernel(zn_blk_ref, zn_all_ref, acc_ref, *, rows, n, nblocks):
    i = pl.program_id(0)
    zb = zn_blk_ref[...]
    za = zn_all_ref[...]
    logits = jax.lax.dot_general(
        zb, za, (((1,), (1,)), ((), ())),
        preferred_element_type=jnp.float32)
    e = jnp.exp(logits * (1.0 / _TEMP))
    col = jax.lax.broadcasted_iota(jnp.int32, (rows, n), 1)
    row = jax.lax.broadcasted_iota(jnp.int32, (rows, n), 0) + i * rows
    is_diag = col == row
    e_all = jnp.sum(jnp.where(is_diag, 0.0, e), axis=1, keepdims=True)

    work = e
    need = jnp.full((rows, 1), _K + 1, jnp.int32)
    topsum = jnp.zeros((rows, 1), jnp.float32)
    maxv = None
    for t in range(_K + 1):
        m = jnp.max(work, axis=1, keepdims=True)
        if t == 0:
            maxv = m
        eqm = work == m
        c = jnp.sum(eqm.astype(jnp.int32), axis=1, keepdims=True)
        take = jnp.minimum(c, need).astype(jnp.float32)
        topsum = topsum + take * m
        need = need - take.astype(jnp.int32)
        if t < _K:
            work = jnp.where(eqm, -jnp.inf, work)
    e_sim = topsum - maxv

    part = jnp.sum(jnp.log(e_all) - jnp.log(e_sim), axis=0, keepdims=True)

    @pl.when(i == 0)
    def _():
        acc_ref[...] = jnp.zeros((1, 1), jnp.float32)

    acc_ref[...] += part

    @pl.when(i == nblocks - 1)
    def _():
        acc_ref[...] = acc_ref[...] / n


def kernel(z):
    n, d = z.shape
    rows = 256
    nblocks = n // rows

    zn = pl.pallas_call(
        _norm_kernel,
        grid=(8,),
        in_specs=[pl.BlockSpec((n // 8, d), lambda i: (i, 0))],
        out_specs=pl.BlockSpec((n // 8, d), lambda i: (i, 0)),
        out_shape=jax.ShapeDtypeStruct((n, d), jnp.float32),
    )(z)

    body = functools.partial(_loss_kernel, rows=rows, n=n, nblocks=nblocks)
    loss = pl.pallas_call(
        body,
        grid=(nblocks,),
        in_specs=[
            pl.BlockSpec((rows, d), lambda i: (i, 0)),
            pl.BlockSpec((n, d), lambda i: (0, 0)),
        ],
        out_specs=pl.BlockSpec((1, 1), lambda i: (0, 0)),
        out_shape=jax.ShapeDtypeStruct((1, 1), jnp.float32),
    )(zn, zn)

    return jnp.reshape(loss, ())


# speedup vs baseline: 26.7758x; 1.0026x over previous
import functools

import jax
import jax.numpy as jnp
from jax.experimental import pallas as pl

_TEMP = 0.5
_K = 10


def _norm_kernel(z_ref, zn_ref):
    z = z_ref[...]
    s = jnp.sum(z * z, axis=1, keepdims=True)
    zn_ref[...] = (z * jax.lax.rsqrt(s)).astype(jnp.bfloat16)


def _loss_kernel(zn_blk_ref, zn_all_ref, acc_ref, *, rows, n, nblocks):
    i = pl.program_id(0)
    zb = zn_blk_ref[...]
    za = zn_all_ref[...]
    logits = jax.lax.dot_general(
        zb, za, (((1,), (1,)), ((), ())),
        preferred_element_type=jnp.float32)
    e = jnp.exp(logits * (1.0 / _TEMP))
    col = jax.lax.broadcasted_iota(jnp.int32, (rows, n), 1)
    row = jax.lax.broadcasted_iota(jnp.int32, (rows, n), 0) + i * rows
    is_diag = col == row
    e_all = jnp.sum(jnp.where(is_diag, 0.0, e), axis=1, keepdims=True)

    work = e
    need = jnp.full((rows, 1), _K + 1, jnp.int32)
    topsum = jnp.zeros((rows, 1), jnp.float32)
    maxv = None
    for t in range(_K + 1):
        m = jnp.max(work, axis=1, keepdims=True)
        if t == 0:
            maxv = m
        eqm = work == m
        c = jnp.sum(eqm.astype(jnp.int32), axis=1, keepdims=True)
        take = jnp.minimum(c, need).astype(jnp.float32)
        topsum = topsum + take * m
        need = need - take.astype(jnp.int32)
        if t < _K:
            work = jnp.where(eqm, -jnp.inf, work)
    e_sim = topsum - maxv

    part = jnp.sum(jnp.log(e_all) - jnp.log(e_sim), axis=0, keepdims=True)

    @pl.when(i == 0)
    def _():
        acc_ref[...] = jnp.zeros((1, 1), jnp.float32)

    acc_ref[...] += part

    @pl.when(i == nblocks - 1)
    def _():
        acc_ref[...] = acc_ref[...] / n


def kernel(z):
    n, d = z.shape
    rows = 256
    nblocks = n // rows

    zn = pl.pallas_call(
        _norm_kernel,
        grid=(8,),
        in_specs=[pl.BlockSpec((n // 8, d), lambda i: (i, 0))],
        out_specs=pl.BlockSpec((n // 8, d), lambda i: (i, 0)),
        out_shape=jax.ShapeDtypeStruct((n, d), jnp.bfloat16),
    )(z)

    body = functools.partial(_loss_kernel, rows=rows, n=n, nblocks=nblocks)
    loss = pl.pallas_call(
        body,
        grid=(nblocks,),
        in_specs=[
            pl.BlockSpec((rows, d), lambda i: (i, 0)),
            pl.BlockSpec((n, d), lambda i: (0, 0)),
        ],
        out_specs=pl.BlockSpec((1, 1), lambda i: (0, 0)),
        out_shape=jax.ShapeDtypeStruct((1, 1), jnp.float32),
    )(zn, zn)

    return jnp.reshape(loss, ())


# speedup vs baseline: 51.1349x; 1.9097x over previous
import functools

import jax
import jax.numpy as jnp
from jax.experimental import pallas as pl

_TEMP = 0.5
_K = 10


def _bitonic_clean_desc(lst):
    n = len(lst)
    if n == 1:
        return lst
    h = n // 2
    hi = [jnp.maximum(lst[i], lst[i + h]) for i in range(h)]
    lo = [jnp.minimum(lst[i], lst[i + h]) for i in range(h)]
    return _bitonic_clean_desc(hi) + _bitonic_clean_desc(lo)


def _sort_desc(lst):
    n = len(lst)
    if n == 1:
        return lst
    a = _sort_desc(lst[: n // 2])
    b = _sort_desc(lst[n // 2:])
    return _bitonic_clean_desc(a + b[::-1])


def _topk_merge(a, b):
    m = [jnp.maximum(a[i], b[15 - i]) for i in range(16)]
    return _bitonic_clean_desc(m)


def _norm_kernel(z_ref, zn_ref):
    z = z_ref[...]
    s = jnp.sum(z * z, axis=1, keepdims=True)
    zn_ref[...] = (z * jax.lax.rsqrt(s)).astype(jnp.bfloat16)


def _loss_kernel(zn_blk_ref, zn_all_ref, acc_ref, *, rows, n, nblocks):
    i = pl.program_id(0)
    zb = zn_blk_ref[...]
    za = zn_all_ref[...]
    logits = jax.lax.dot_general(
        zb, za, (((1,), (1,)), ((), ())),
        preferred_element_type=jnp.float32)
    e = jnp.exp(logits * (1.0 / _TEMP))
    col = jax.lax.broadcasted_iota(jnp.int32, (rows, n), 1)
    row = jax.lax.broadcasted_iota(jnp.int32, (rows, n), 0) + i * rows
    is_diag = col == row
    e_all = jnp.sum(jnp.where(is_diag, 0.0, e), axis=1, keepdims=True)

    slices = [e[:, g * 128:(g + 1) * 128] for g in range(64)]
    runs = [_sort_desc(slices[i * 16:(i + 1) * 16]) for i in range(4)]
    ab = _topk_merge(runs[0], runs[1])
    cd = _topk_merge(runs[2], runs[3])
    cand = [jnp.maximum(ab[i], cd[15 - i]) for i in range(16)]
    work = jnp.concatenate(cand, axis=1)

    need = jnp.full((rows, 1), _K + 1, jnp.int32)
    topsum = jnp.zeros((rows, 1), jnp.float32)
    maxv = None
    for t in range(_K + 1):
        m = jnp.max(work, axis=1, keepdims=True)
        if t == 0:
            maxv = m
        eqm = work == m
        c = jnp.sum(eqm.astype(jnp.int32), axis=1, keepdims=True)
        take = jnp.minimum(c, need).astype(jnp.float32)
        topsum = topsum + take * m
        need = need - take.astype(jnp.int32)
        if t < _K:
            work = jnp.where(eqm, -jnp.inf, work)
    e_sim = topsum - maxv

    part = jnp.sum(jnp.log(e_all) - jnp.log(e_sim), axis=0, keepdims=True)

    @pl.when(i == 0)
    def _():
        acc_ref[...] = jnp.zeros((1, 1), jnp.float32)

    acc_ref[...] += part

    @pl.when(i == nblocks - 1)
    def _():
        acc_ref[...] = acc_ref[...] / n


def kernel(z):
    n, d = z.shape
    rows = 256
    nblocks = n // rows

    zn = pl.pallas_call(
        _norm_kernel,
        grid=(8,),
        in_specs=[pl.BlockSpec((n // 8, d), lambda i: (i, 0))],
        out_specs=pl.BlockSpec((n // 8, d), lambda i: (i, 0)),
        out_shape=jax.ShapeDtypeStruct((n, d), jnp.bfloat16),
    )(z)

    body = functools.partial(_loss_kernel, rows=rows, n=n, nblocks=nblocks)
    loss = pl.pallas_call(
        body,
        grid=(nblocks,),
        in_specs=[
            pl.BlockSpec((rows, d), lambda i: (i, 0)),
            pl.BlockSpec((n, d), lambda i: (0, 0)),
        ],
        out_specs=pl.BlockSpec((1, 1), lambda i: (0, 0)),
        out_shape=jax.ShapeDtypeStruct((1, 1), jnp.float32),
    )(zn, zn)

    return jnp.reshape(loss, ())
